# baseline (speedup 1.0000x reference)
import functools

import jax
import jax.numpy as jnp
from jax import lax
from jax.experimental import pallas as pl
from jax.experimental.pallas import tpu as pltpu
from jax.experimental.pallas import tpu_sc as plsc

_NUM_ACTIONS = 100000
_DIM = 64
_BATCH = 16384

_NC, _NS = 2, 16
_NW = _NC * _NS
_BPW = _BATCH // _NW
_LANES = 16


def _gather_body(actions_hbm, table2_hbm, outt_hbm, idx_v, pair_v, outt_v, sem):
    wid = lax.axis_index("s") * _NC + lax.axis_index("c")
    base = wid * _BPW
    pltpu.sync_copy(actions_hbm.at[pl.ds(base, _BPW)], idx_v)

    def chunk(c, carry):
        vec = idx_v[pl.ds(c * _LANES, _LANES)]
        for l in range(_LANES):
            pltpu.async_copy(
                table2_hbm.at[vec[l] >> 1], pair_v.at[c * _LANES + l], sem
            )
        return carry

    lax.fori_loop(0, _BPW // _LANES, chunk, 0)

    def tchunk(c, carry):
        pltpu.make_async_copy(
            table2_hbm.at[pl.ds(0, _LANES)],
            pair_v.at[pl.ds(c * _LANES, _LANES)],
            sem,
        ).wait()
        i_vec = lax.broadcasted_iota(jnp.int32, (_LANES,), 0) + c * _LANES
        half = (idx_v[pl.ds(c * _LANES, _LANES)] & 1) << 6
        for f in range(_DIM):
            f_vec = jnp.full((_LANES,), f, jnp.int32)
            vals = plsc.load_gather(pair_v, [i_vec, half + f])
            plsc.store_scatter(outt_v, [f_vec, i_vec], vals)
        return carry

    lax.fori_loop(0, _BPW // _LANES, tchunk, 0)
    pltpu.sync_copy(outt_v, outt_hbm.at[:, pl.ds(base, _BPW)])


def kernel(actions, embedding_weight):
    actions = actions.astype(jnp.int32)
    table2 = jnp.reshape(embedding_weight, (_NUM_ACTIONS // 2, 2 * _DIM))
    mesh = plsc.VectorSubcoreMesh(core_axis_name="c", subcore_axis_name="s")
    run = pl.kernel(
        _gather_body,
        mesh=mesh,
        compiler_params=pltpu.CompilerParams(needs_layout_passes=False),
        out_type=jax.ShapeDtypeStruct((_DIM, _BATCH), jnp.float32),
        scratch_types=[
            pltpu.VMEM((_BPW,), jnp.int32),
            pltpu.VMEM((_BPW, 2 * _DIM), jnp.float32),
            pltpu.VMEM((_DIM, _BPW), jnp.float32),
            pltpu.SemaphoreType.DMA,
        ],
    )
    out_t = run(actions, table2)
    return jnp.transpose(out_t)
